# baseline (speedup 1.0000x reference)
import functools

import jax
import jax.numpy as jnp
from jax import lax
from jax.experimental import pallas as pl
from jax.experimental.pallas import tpu as pltpu
from jax.experimental.pallas import tpu_sc as plsc

BATCH = 1024
HIST = 200
EMBED = 128

NC = 2
NS = 16
NW = NC * NS
N = BATCH * HIST
CHUNK = 128
NCH = N // (NW * CHUNK)
NBUF = 3
NGRP = NCH // NBUF

_mesh = plsc.VectorSubcoreMesh(core_axis_name="c", subcore_axis_name="s")


@functools.partial(
    pl.kernel,
    out_type=jax.ShapeDtypeStruct((NW, NCH, CHUNK, EMBED), jnp.float32),
    mesh=_mesh,
    scratch_types=[
        pltpu.VMEM((NCH, CHUNK), jnp.int32),
        [pltpu.VMEM((CHUNK, EMBED), jnp.float32) for _ in range(NBUF)],
        pltpu.VMEM((CHUNK, EMBED), jnp.float32),
        [pltpu.SemaphoreType.DMA for _ in range(NBUF)],
        [pltpu.SemaphoreType.DMA for _ in range(NBUF)],
    ],
)
def _gather_kernel(idx_hbm, table_hbm, out_hbm, idx_v, bufs, wsrc, gsems, wsems):
    wid = lax.axis_index("s") * NC + lax.axis_index("c")
    pltpu.sync_copy(idx_hbm.at[wid], idx_v)
    out_w = out_hbm.at[wid]

    def gather_start(j, b):
        pltpu.async_copy(table_hbm.at[idx_v.at[j]], bufs[b], gsems[b])

    def gather_wait(j, b):
        pltpu.make_async_copy(table_hbm.at[idx_v.at[j]], bufs[b], gsems[b]).wait()

    def wb_start(j, b):
        pltpu.async_copy(wsrc, out_w.at[j], wsems[b])

    def wb_wait(j, b):
        pltpu.make_async_copy(wsrc, out_w.at[j], wsems[b]).wait()

    for b in range(NBUF):
        gather_start(b, b)
        wb_start(b, b)

    def outer(i, carry):
        for b in range(NBUF):
            j = i * NBUF + b
            gather_wait(j, b)
            wb_wait(j, b)

            @pl.when(j + NBUF < NCH)
            def _():
                gather_start(j + NBUF, b)
                wb_start(j + NBUF, b)

        return carry

    lax.fori_loop(0, NGRP, outer, 0)

    for j in range(NGRP * NBUF, NCH):
        gather_wait(j, j % NBUF)
        wb_wait(j, j % NBUF)


def kernel(input, table):
    idx = input.reshape(NW, NCH, CHUNK).astype(jnp.int32)
    out = _gather_kernel(idx, table)
    return out.reshape(BATCH, HIST, EMBED)
